# baseline (speedup 1.0000x reference)
import functools

import jax
from jax import lax
import jax.numpy as jnp
from jax.experimental import pallas as pl
from jax.experimental.pallas import tpu as pltpu
from jax.experimental.pallas import tpu_sc as plsc

TEMP = 0.05

B = 1024
D = 256
K = 8192
KBLK = 1024
NSTEPS = K // KBLK
LANES = 128
GRPS = KBLK // LANES

SC_NC = 2
SC_NS = 16
SC_NW = SC_NC * SC_NS
B_PER_W = B // SC_NW


def _gather_body(table_hbm, idx_hbm, out_hbm, idx_v, rows_v, sem):
    wid = lax.axis_index("s") * SC_NC + lax.axis_index("c")
    base = wid * B_PER_W
    pltpu.sync_copy(idx_hbm.at[pl.ds(base, B_PER_W)], idx_v)
    pltpu.async_copy(table_hbm.at[idx_v], rows_v, sem).wait()
    pltpu.sync_copy(rows_v, out_hbm.at[pl.ds(base, B_PER_W)])


@functools.cache
def _gather_targets():
    return pl.kernel(
        _gather_body,
        out_type=jax.ShapeDtypeStruct((B, D), jnp.float32),
        mesh=plsc.VectorSubcoreMesh(core_axis_name="c", subcore_axis_name="s"),
        scratch_types=[
            pltpu.VMEM((B_PER_W,), jnp.int32),
            pltpu.VMEM((B_PER_W, D), jnp.float32),
            pltpu.SemaphoreType.DMA,
        ],
    )


def _lane_sum(a):
    n = a.shape[1] // LANES
    parts = [a[:, g * LANES:(g + 1) * LANES] for g in range(n)]
    while len(parts) > 1:
        parts = [parts[i] + parts[i + 1] for i in range(0, len(parts), 2)]
    return parts[0]


def _loss_body(x_ref, f_ref, g_ref, out_ref, xs_ref, inv_ref, acc_ref):
    k = pl.program_id(0)

    @pl.when(k == 0)
    def _init():
        x = x_ref[...]
        norm = jnp.sqrt(jnp.sum(x * x, axis=1, keepdims=True))
        inv = 1.0 / (jnp.maximum(norm, 1e-12) * TEMP)
        inv_ref[...] = inv
        xs_ref[...] = (x * inv).astype(jnp.bfloat16)
        acc_ref[...] = jnp.zeros_like(acc_ref)

    s = jax.lax.dot_general(
        xs_ref[...], f_ref[...].astype(jnp.bfloat16),
        dimension_numbers=(((1,), (1,)), ((), ())),
        preferred_element_type=jnp.float32,
    )
    acc_ref[...] += _lane_sum(jnp.exp(s))

    @pl.when(k == NSTEPS - 1)
    def _fini():
        lse = jnp.log(jnp.sum(acc_ref[...], axis=1, keepdims=True))
        tgt = jnp.sum(_lane_sum(x_ref[...] * g_ref[...]), axis=1,
                      keepdims=True) * inv_ref[...]
        out_ref[...] = jnp.mean(lse - tgt, keepdims=True).reshape(1, 1)


def _tc_loss(inputs, features, gathered):
    out = pl.pallas_call(
        _loss_body,
        grid=(NSTEPS,),
        in_specs=[
            pl.BlockSpec((B, D), lambda k: (0, 0)),
            pl.BlockSpec((KBLK, D), lambda k: (k, 0)),
            pl.BlockSpec((B, D), lambda k: (0, 0)),
        ],
        out_specs=pl.BlockSpec((1, 1), lambda k: (0, 0)),
        out_shape=jax.ShapeDtypeStruct((1, 1), jnp.float32),
        scratch_shapes=[
            pltpu.VMEM((B, D), jnp.bfloat16),
            pltpu.VMEM((B, 1), jnp.float32),
            pltpu.VMEM((B, LANES), jnp.float32),
        ],
    )(inputs, features, gathered)
    return out[0, 0]


@jax.jit
def _run(inputs, targets, features):
    g = _gather_targets()(features, targets.astype(jnp.int32))
    return _tc_loss(inputs, features, g)


def kernel(inputs, targets, features):
    return _run(inputs, targets, features)


# speedup vs baseline: 1.1172x; 1.1172x over previous
import functools

import jax
from jax import lax
import jax.numpy as jnp
from jax.experimental import pallas as pl
from jax.experimental.pallas import tpu as pltpu
from jax.experimental.pallas import tpu_sc as plsc

TEMP = 0.05

B = 1024
D = 256
K = 8192
KBLK = 1024
NSTEPS = K // KBLK
LANES = 128
GRPS = KBLK // LANES

SC_NC = 2
SC_NS = 16
SC_NW = SC_NC * SC_NS
B_PER_W = B // SC_NW


def _gather_body(table_hbm, idx_hbm, out_hbm, idx_v, rows_v, sem):
    wid = lax.axis_index("s") * SC_NC + lax.axis_index("c")
    base = wid * B_PER_W
    pltpu.sync_copy(idx_hbm.at[pl.ds(base, B_PER_W)], idx_v)
    pltpu.async_copy(table_hbm.at[idx_v], rows_v, sem).wait()
    pltpu.sync_copy(rows_v, out_hbm.at[pl.ds(base, B_PER_W)])


@functools.cache
def _gather_targets():
    return pl.kernel(
        _gather_body,
        out_type=jax.ShapeDtypeStruct((B, D), jnp.float32),
        mesh=plsc.VectorSubcoreMesh(core_axis_name="c", subcore_axis_name="s"),
        scratch_types=[
            pltpu.VMEM((B_PER_W,), jnp.int32),
            pltpu.VMEM((B_PER_W, D), jnp.float32),
            pltpu.SemaphoreType.DMA,
        ],
    )


def _lane_sum(a):
    n = a.shape[1] // LANES
    parts = [a[:, g * LANES:(g + 1) * LANES] for g in range(n)]
    while len(parts) > 1:
        parts = [parts[i] + parts[i + 1] for i in range(0, len(parts), 2)]
    return parts[0]


def _sumexp_body(x_ref, f_ref, acc_out_ref, xs_ref):
    k = pl.program_id(0)

    @pl.when(k == 0)
    def _init():
        x = x_ref[...]
        norm = jnp.sqrt(jnp.sum(x * x, axis=1, keepdims=True))
        xs_ref[...] = (x * (1.0 / (jnp.maximum(norm, 1e-12) * TEMP))
                       ).astype(jnp.bfloat16)
        acc_out_ref[...] = jnp.zeros_like(acc_out_ref)

    s = jax.lax.dot_general(
        xs_ref[...], f_ref[...].astype(jnp.bfloat16),
        dimension_numbers=(((1,), (1,)), ((), ())),
        preferred_element_type=jnp.float32,
    )
    acc_out_ref[...] += _lane_sum(jnp.exp(s))


def _combine_body(x_ref, g_ref, acc_ref, out_ref):
    x = x_ref[...]
    norm = jnp.sqrt(jnp.sum(x * x, axis=1, keepdims=True))
    inv = 1.0 / (jnp.maximum(norm, 1e-12) * TEMP)
    tgt = jnp.sum(_lane_sum(x * g_ref[...]), axis=1, keepdims=True) * inv
    lse = jnp.log(jnp.sum(acc_ref[...], axis=1, keepdims=True))
    out_ref[...] = jnp.mean(lse - tgt, keepdims=True).reshape(1, 1)


def _sumexp(inputs, features):
    return pl.pallas_call(
        _sumexp_body,
        grid=(NSTEPS,),
        in_specs=[
            pl.BlockSpec((B, D), lambda k: (0, 0)),
            pl.BlockSpec((KBLK, D), lambda k: (k, 0)),
        ],
        out_specs=pl.BlockSpec((B, LANES), lambda k: (0, 0)),
        out_shape=jax.ShapeDtypeStruct((B, LANES), jnp.float32),
        scratch_shapes=[pltpu.VMEM((B, D), jnp.bfloat16)],
    )(inputs, features)


def _combine(inputs, gathered, acc):
    out = pl.pallas_call(
        _combine_body,
        out_shape=jax.ShapeDtypeStruct((1, 1), jnp.float32),
    )(inputs, gathered, acc)
    return out[0, 0]


@jax.jit
def _run(inputs, targets, features):
    g = _gather_targets()(features, targets.astype(jnp.int32))
    acc = _sumexp(inputs, features)
    return _combine(inputs, g, acc)


def kernel(inputs, targets, features):
    return _run(inputs, targets, features)


# speedup vs baseline: 1.8245x; 1.6331x over previous
import functools

import jax
import jax.numpy as jnp
from jax.experimental import pallas as pl
from jax.experimental.pallas import tpu as pltpu

TEMP = 0.05

B = 1024
D = 256
K = 8192
KBLK = 1024
NPAR = 2
NSEQ = K // KBLK // NPAR
LANES = 128


def _lane_sum(a):
    n = a.shape[1] // LANES
    parts = [a[:, g * LANES:(g + 1) * LANES] for g in range(n)]
    while len(parts) > 1:
        parts = [parts[i] + parts[i + 1] for i in range(0, len(parts), 2)]
    return parts[0]


def _sumexp_body(x_ref, t_ref, f_ref, acc_ref, tgt_ref, xs_ref):
    c = pl.program_id(0)
    j = pl.program_id(1)

    @pl.when(j == 0)
    def _init():
        x = x_ref[...]
        norm = jnp.sqrt(jnp.sum(x * x, axis=1, keepdims=True))
        xs_ref[...] = (x * (1.0 / (jnp.maximum(norm, 1e-12) * TEMP))
                       ).astype(jnp.bfloat16)
        acc_ref[...] = jnp.zeros_like(acc_ref)
        tgt_ref[...] = jnp.zeros_like(tgt_ref)

    s = jax.lax.dot_general(
        xs_ref[...], f_ref[...].astype(jnp.bfloat16),
        dimension_numbers=(((1,), (1,)), ((), ())),
        preferred_element_type=jnp.float32,
    )
    acc_ref[0] += _lane_sum(jnp.exp(s))

    cols = (c * NSEQ + j) * KBLK + jax.lax.broadcasted_iota(
        jnp.int32, (B, KBLK), 1)
    tgt_ref[0] += _lane_sum(jnp.where(cols == t_ref[...], s, 0.0))


def _combine_body(acc_ref, tgt_ref, out_ref):
    a = acc_ref[0]
    t = tgt_ref[0]
    for c in range(1, NPAR):
        a = a + acc_ref[c]
        t = t + tgt_ref[c]
    lse = jnp.log(jnp.sum(a, axis=1, keepdims=True))
    tgt = jnp.sum(t, axis=1, keepdims=True)
    out_ref[...] = jnp.mean(lse - tgt, keepdims=True).reshape(1, 1)


def _sumexp(inputs, t2d, features):
    return pl.pallas_call(
        _sumexp_body,
        grid=(NPAR, NSEQ),
        in_specs=[
            pl.BlockSpec((B, D), lambda c, j: (0, 0)),
            pl.BlockSpec((B, 1), lambda c, j: (0, 0)),
            pl.BlockSpec((KBLK, D), lambda c, j: (c * NSEQ + j, 0)),
        ],
        out_specs=[
            pl.BlockSpec((1, B, LANES), lambda c, j: (c, 0, 0)),
            pl.BlockSpec((1, B, LANES), lambda c, j: (c, 0, 0)),
        ],
        out_shape=[
            jax.ShapeDtypeStruct((NPAR, B, LANES), jnp.float32),
            jax.ShapeDtypeStruct((NPAR, B, LANES), jnp.float32),
        ],
        scratch_shapes=[pltpu.VMEM((B, D), jnp.bfloat16)],
        compiler_params=pltpu.CompilerParams(
            dimension_semantics=("parallel", "arbitrary")),
    )(inputs, t2d, features)


def _combine(acc, tgt):
    out = pl.pallas_call(
        _combine_body,
        out_shape=jax.ShapeDtypeStruct((1, 1), jnp.float32),
    )(acc, tgt)
    return out[0, 0]


@jax.jit
def _run(inputs, targets, features):
    t2d = targets.astype(jnp.int32).reshape(B, 1)
    acc, tgt = _sumexp(inputs, t2d, features)
    return _combine(acc, tgt)


def kernel(inputs, targets, features):
    return _run(inputs, targets, features)


# speedup vs baseline: 2.3135x; 1.2681x over previous
import math

import jax
import jax.numpy as jnp
from jax.experimental import pallas as pl
from jax.experimental.pallas import tpu as pltpu

TEMP = 0.05
LOG2E = math.log2(math.e)
LN2 = math.log(2.0)

B = 1024
D = 256
K = 8192
KBLK = 1024
NSTEPS = K // KBLK
LANES = 128


def _lane_sum(a):
    n = a.shape[1] // LANES
    parts = [a[:, g * LANES:(g + 1) * LANES] for g in range(n)]
    while len(parts) > 1:
        parts = [parts[i] + parts[i + 1] for i in range(0, len(parts), 2)]
    return parts[0]


def _loss_kernel(x_ref, t_ref, f_ref, out_ref, xs_ref, acc_ref, tgt_ref):
    k = pl.program_id(0)

    @pl.when(k == 0)
    def _init():
        x = x_ref[...]
        norm = jnp.sqrt(jnp.sum(x * x, axis=1, keepdims=True))
        scale = LOG2E / (jnp.maximum(norm, 1e-12) * TEMP)
        xs_ref[...] = (x * scale).astype(jnp.bfloat16)
        acc_ref[...] = jnp.zeros_like(acc_ref)
        tgt_ref[...] = jnp.zeros_like(tgt_ref)

    s2 = jax.lax.dot_general(
        xs_ref[...], f_ref[...].astype(jnp.bfloat16),
        dimension_numbers=(((1,), (1,)), ((), ())),
        preferred_element_type=jnp.float32,
    )
    acc_ref[...] += _lane_sum(jnp.exp2(s2))

    cols = k * KBLK + jax.lax.broadcasted_iota(jnp.int32, (B, KBLK), 1)
    tgt_ref[...] += _lane_sum(jnp.where(cols == t_ref[...], s2, 0.0))

    @pl.when(k == NSTEPS - 1)
    def _fini():
        lse = jnp.log(jnp.sum(acc_ref[...], axis=1, keepdims=True))
        tgt = jnp.sum(tgt_ref[...], axis=1, keepdims=True) * LN2
        out_ref[...] = jnp.mean(lse - tgt, keepdims=True).reshape(1, 1)


@jax.jit
def _run(inputs, targets, features):
    t2d = targets.astype(jnp.int32).reshape(B, 1)
    out = pl.pallas_call(
        _loss_kernel,
        grid=(NSTEPS,),
        in_specs=[
            pl.BlockSpec((B, D), lambda k: (0, 0)),
            pl.BlockSpec((B, 1), lambda k: (0, 0)),
            pl.BlockSpec((KBLK, D), lambda k: (k, 0)),
        ],
        out_specs=pl.BlockSpec((1, 1), lambda k: (0, 0)),
        out_shape=jax.ShapeDtypeStruct((1, 1), jnp.float32),
        scratch_shapes=[
            pltpu.VMEM((B, D), jnp.bfloat16),
            pltpu.VMEM((B, LANES), jnp.float32),
            pltpu.VMEM((B, LANES), jnp.float32),
        ],
    )(inputs, t2d, features)
    return out[0, 0]


def kernel(inputs, targets, features):
    return _run(inputs, targets, features)


# speedup vs baseline: 2.4629x; 1.0646x over previous
import math

import jax
import jax.numpy as jnp
from jax.experimental import pallas as pl
from jax.experimental.pallas import tpu as pltpu

TEMP = 0.05
LOG2E = math.log2(math.e)
LN2 = math.log(2.0)

B = 1024
D = 256
K = 8192
KBLK = 8192
NSTEPS = K // KBLK
LANES = 128


def _lane_sum(a):
    n = a.shape[1] // LANES
    parts = [a[:, g * LANES:(g + 1) * LANES] for g in range(n)]
    while len(parts) > 1:
        parts = [parts[i] + parts[i + 1] for i in range(0, len(parts), 2)]
    return parts[0]


def _loss_kernel(x_ref, t_ref, f_ref, out_ref, xs_ref, acc_ref, tgt_ref):
    k = pl.program_id(0)

    @pl.when(k == 0)
    def _init():
        x = x_ref[...]
        norm = jnp.sqrt(jnp.sum(x * x, axis=1, keepdims=True))
        scale = LOG2E / (jnp.maximum(norm, 1e-12) * TEMP)
        xs_ref[...] = (x * scale).astype(jnp.bfloat16)
        acc_ref[...] = jnp.zeros_like(acc_ref)
        tgt_ref[...] = jnp.zeros_like(tgt_ref)

    s2 = jax.lax.dot_general(
        xs_ref[...], f_ref[...].astype(jnp.bfloat16),
        dimension_numbers=(((1,), (1,)), ((), ())),
        preferred_element_type=jnp.float32,
    )
    acc_ref[...] += _lane_sum(jnp.exp2(s2))

    cols = k * KBLK + jax.lax.broadcasted_iota(jnp.int32, (B, KBLK), 1)
    tgt_ref[...] += _lane_sum(jnp.where(cols == t_ref[...], s2, 0.0))

    @pl.when(k == NSTEPS - 1)
    def _fini():
        lse = jnp.log(jnp.sum(acc_ref[...], axis=1, keepdims=True))
        tgt = jnp.sum(tgt_ref[...], axis=1, keepdims=True) * LN2
        out_ref[...] = jnp.mean(lse - tgt, keepdims=True).reshape(1, 1)


@jax.jit
def _run(inputs, targets, features):
    t2d = targets.astype(jnp.int32).reshape(B, 1)
    out = pl.pallas_call(
        _loss_kernel,
        grid=(NSTEPS,),
        in_specs=[
            pl.BlockSpec((B, D), lambda k: (0, 0)),
            pl.BlockSpec((B, 1), lambda k: (0, 0)),
            pl.BlockSpec((KBLK, D), lambda k: (k, 0)),
        ],
        out_specs=pl.BlockSpec((1, 1), lambda k: (0, 0)),
        out_shape=jax.ShapeDtypeStruct((1, 1), jnp.float32),
        scratch_shapes=[
            pltpu.VMEM((B, D), jnp.bfloat16),
            pltpu.VMEM((B, LANES), jnp.float32),
            pltpu.VMEM((B, LANES), jnp.float32),
        ],
    )(inputs, t2d, features)
    return out[0, 0]


def kernel(inputs, targets, features):
    return _run(inputs, targets, features)
